# baseline (speedup 1.0000x reference)
import dataclasses
import functools

import jax
import jax.numpy as jnp
from jax import lax
from jax.experimental import pallas as pl
from jax.experimental.pallas import tpu as pltpu
from jax.experimental.pallas import tpu_sc as plsc

N = 10000
D = 128
H = 128
E = 320000

NC = 2
NS = 16
NW = NC * NS
K = 128
CPW = 80
G = 8
NG = CPW // G
E_PAD = NW * CPW * K
N_PAD = 10240
NR = N_PAD // 128
RPS = N_PAD // NS

_mesh = plsc.VectorSubcoreMesh(core_axis_name="c", subcore_axis_name="s")

_sc_params = pltpu.CompilerParams()
if "needs_layout_passes" in pltpu.CompilerParams.__dataclass_fields__:
  _sc_params = dataclasses.replace(_sc_params, needs_layout_passes=False)


@functools.partial(
    pl.kernel,
    out_type=jax.ShapeDtypeStruct((NC, N_PAD, H), jnp.float32),
    mesh=_mesh,
    scratch_types=[
        pltpu.VMEM((CPW, K), jnp.int32),
        pltpu.VMEM((CPW, K), jnp.int32),
        pltpu.VMEM((K, H), jnp.float32),
        pltpu.VMEM_SHARED((N_PAD, H), jnp.float32),
        pltpu.SemaphoreType.DMA,
    ],
)
def _agg128(table_hbm, src_hbm, dst_hbm, zeros_hbm, out_hbm,
            src_v, dst_v, rows_v, acc_sh, gsem):
  core = lax.axis_index("c")
  sub = lax.axis_index("s")
  w = core * NS + sub
  pltpu.sync_copy(zeros_hbm, acc_sh.at[pl.ds(sub * RPS, RPS)])
  pltpu.sync_copy(src_hbm.at[w], src_v)
  pltpu.sync_copy(dst_hbm.at[w], dst_v)
  plsc.subcore_barrier()

  @pl.loop(0, CPW)
  def _(j):
    pltpu.async_copy(table_hbm.at[src_v.at[j]], rows_v, gsem).wait()
    pltpu.sync_copy(rows_v, acc_sh.at[dst_v.at[j]], add=True)

  plsc.subcore_barrier()
  pltpu.sync_copy(acc_sh.at[pl.ds(sub * RPS, RPS)],
                  out_hbm.at[core, pl.ds(sub * RPS, RPS)])


@functools.partial(
    pl.kernel,
    out_type=jax.ShapeDtypeStruct((NW, NR, 128), jnp.float32),
    mesh=_mesh,
    scratch_types=[
        pltpu.VMEM((CPW, K), jnp.int32),
        pltpu.VMEM((NR, 128), jnp.float32),
    ],
    compiler_params=_sc_params,
)
def _hist(dst_hbm, zeros_hbm, out_hbm, dst_v, acc_v):
  core = lax.axis_index("c")
  sub = lax.axis_index("s")
  w = core * NS + sub
  pltpu.sync_copy(zeros_hbm, acc_v)
  pltpu.sync_copy(dst_hbm.at[w], dst_v)
  ones = jnp.full((16,), 1.0, jnp.float32)

  @pl.loop(0, CPW)
  def _(j):
    @pl.loop(0, K, step=16)
    def _(t):
      d = dst_v[j, pl.ds(t, 16)]
      plsc.addupdate_scatter(
          acc_v, [jax.lax.shift_right_logical(d, 7), d & 127], ones)

  pltpu.sync_copy(acc_v, out_hbm.at[w])


@functools.partial(
    pl.kernel,
    out_type=jax.ShapeDtypeStruct((NW, NR, 128), jnp.float32),
    mesh=_mesh,
    scratch_types=[
        pltpu.VMEM((CPW, K), jnp.int32),
        pltpu.VMEM((CPW, K), jnp.int32),
        pltpu.VMEM((NR, 128), jnp.float32),
        pltpu.VMEM((NR, 128), jnp.float32),
    ],
    compiler_params=_sc_params,
)
def _sagg(z_hbm, src_hbm, dst_hbm, zeros_hbm, out_hbm,
          src_v, dst_v, z_v, acc_v):
  core = lax.axis_index("c")
  sub = lax.axis_index("s")
  w = core * NS + sub
  pltpu.sync_copy(zeros_hbm, acc_v)
  pltpu.sync_copy(z_hbm, z_v)
  pltpu.sync_copy(src_hbm.at[w], src_v)
  pltpu.sync_copy(dst_hbm.at[w], dst_v)

  @pl.loop(0, CPW)
  def _(j):
    @pl.loop(0, K, step=16)
    def _(t):
      s = src_v[j, pl.ds(t, 16)]
      d = dst_v[j, pl.ds(t, 16)]
      vals = plsc.load_gather(
          z_v, [jax.lax.shift_right_logical(s, 7), s & 127])
      plsc.addupdate_scatter(
          acc_v, [jax.lax.shift_right_logical(d, 7), d & 127], vals)

  pltpu.sync_copy(acc_v, out_hbm.at[w])


RB = 2000


def _mm1_body(x_ref, w_ref, o_ref):
  x = x_ref[...]
  w = w_ref[...]
  xh = x.astype(jnp.bfloat16).astype(jnp.float32)
  xl = x - xh
  wh = w.astype(jnp.bfloat16).astype(jnp.float32)
  wl = w - wh
  dims = (((1,), (1,)), ((), ()))
  dot = functools.partial(
      lax.dot_general, dimension_numbers=dims,
      preferred_element_type=jnp.float32)
  o_ref[...] = dot(xh, wh) + dot(xh, wl) + dot(xl, wh)


_mm1 = pl.pallas_call(
    _mm1_body,
    grid=(N // RB,),
    in_specs=[pl.BlockSpec((RB, D), lambda i: (i, 0)),
              pl.BlockSpec((H, D), lambda i: (0, 0))],
    out_specs=pl.BlockSpec((RB, H), lambda i: (i, 0)),
    out_shape=jax.ShapeDtypeStruct((N, H), jnp.float32),
)


def _dinv_body(degp_ref, o_ref):
  deg = jnp.full((NR, 128), 1.0, jnp.float32)
  for i in range(NW):
    deg = deg + degp_ref[i]
  o_ref[...] = lax.rsqrt(deg)


_dinv = pl.pallas_call(
    _dinv_body,
    grid=(1,),
    in_specs=[pl.BlockSpec((NW, NR, 128), lambda i: (0, 0, 0))],
    out_specs=pl.BlockSpec((NR, 128), lambda i: (0, 0)),
    out_shape=jax.ShapeDtypeStruct((NR, 128), jnp.float32),
)


def _scale1_body(xw_ref, dinv_ref, o_y):
  o_y[...] = xw_ref[...] * dinv_ref[...]


_scale1 = pl.pallas_call(
    _scale1_body,
    grid=(N // RB,),
    in_specs=[pl.BlockSpec((RB, H), lambda i: (i, 0)),
              pl.BlockSpec((RB, 1), lambda i: (i, 0))],
    out_specs=pl.BlockSpec((RB, H), lambda i: (i, 0)),
    out_shape=jax.ShapeDtypeStruct((N, H), jnp.float32),
)


def _layer2_body(accp_ref, y_ref, dinv_ref, b1_ref, w2_ref, o_z):
  s = accp_ref[0] + accp_ref[1] + y_ref[...]
  h = jnp.maximum(s * dinv_ref[...] + b1_ref[...][None, :], 0.0)
  o_z[...] = jnp.sum(h * w2_ref[...], axis=1, keepdims=True) * dinv_ref[...]


_layer2 = pl.pallas_call(
    _layer2_body,
    grid=(N // RB,),
    in_specs=[pl.BlockSpec((2, RB, H), lambda i: (0, i, 0)),
              pl.BlockSpec((RB, H), lambda i: (i, 0)),
              pl.BlockSpec((RB, 1), lambda i: (i, 0)),
              pl.BlockSpec((H,), lambda i: (0,)),
              pl.BlockSpec((1, H), lambda i: (0, 0))],
    out_specs=pl.BlockSpec((RB, 1), lambda i: (i, 0)),
    out_shape=jax.ShapeDtypeStruct((N, 1), jnp.float32),
)


def _final_body(acc2p_ref, z_ref, dinv_ref, b2_ref, o_ref):
  s = z_ref[...]
  for i in range(NW):
    s = s + acc2p_ref[i]
  o_ref[...] = dinv_ref[...] * s + b2_ref[...]


_final = pl.pallas_call(
    _final_body,
    grid=(1,),
    in_specs=[pl.BlockSpec((NW, NR, 128), lambda i: (0, 0, 0)),
              pl.BlockSpec((NR, 128), lambda i: (0, 0)),
              pl.BlockSpec((NR, 128), lambda i: (0, 0)),
              pl.BlockSpec((1,), lambda i: (0,))],
    out_specs=pl.BlockSpec((NR, 128), lambda i: (0, 0)),
    out_shape=jax.ShapeDtypeStruct((NR, 128), jnp.float32),
)


def kernel(x, edge_index, W1, b1, W2, b2):
  src = edge_index[0]
  dst = edge_index[1]
  pad_idx = N + jnp.arange(E_PAD - E, dtype=jnp.int32) % (N_PAD - N)
  pad_src = pad_idx
  pad_dst = pad_idx
  src3 = jnp.concatenate([src, pad_src]).reshape(NW, CPW, K)
  dst3 = jnp.concatenate([dst, pad_dst]).reshape(NW, CPW, K)

  zeros_nr = jnp.zeros((NR, 128), jnp.float32)
  zeros128 = jnp.zeros((RPS, H), jnp.float32)

  degp = _hist(dst3, zeros_nr)
  xw = _mm1(x, W1)
  dinv80 = _dinv(degp)
  dinv = dinv80.reshape(N_PAD, 1)[:N]
  y = _scale1(xw, dinv)
  y_pad = jnp.concatenate(
      [y, jnp.zeros((N_PAD - N, H), jnp.float32)], axis=0)
  accp = _agg128(y_pad, src3, dst3, zeros128)
  z = _layer2(accp[:, :N, :], y, dinv, b1, W2)
  z80 = jnp.concatenate(
      [z[:, 0], jnp.zeros((N_PAD - N,), jnp.float32)]).reshape(NR, 128)
  acc2p = _sagg(z80, src3, dst3, zeros_nr)
  out80 = _final(acc2p, z80, dinv80, b2)
  return out80.reshape(N_PAD)[:N]


# speedup vs baseline: 1.0988x; 1.0988x over previous
import dataclasses
import functools

import jax
import jax.numpy as jnp
from jax import lax
from jax.experimental import pallas as pl
from jax.experimental.pallas import tpu as pltpu
from jax.experimental.pallas import tpu_sc as plsc

N = 10000
D = 128
H = 128
E = 320000

NC = 2
NS = 16
NW = NC * NS
K = 128
CPW = 80
G = 8
NG = CPW // G
E_PAD = NW * CPW * K
N_PAD = 10240
NR = N_PAD // 128
RPS = N_PAD // NS

_mesh = plsc.VectorSubcoreMesh(core_axis_name="c", subcore_axis_name="s")

_sc_params = pltpu.CompilerParams()
if "needs_layout_passes" in pltpu.CompilerParams.__dataclass_fields__:
  _sc_params = dataclasses.replace(_sc_params, needs_layout_passes=False)


@functools.partial(
    pl.kernel,
    out_type=jax.ShapeDtypeStruct((NC, N_PAD, H), jnp.float32),
    mesh=_mesh,
    scratch_types=[
        pltpu.VMEM((CPW // 2, K), jnp.int32),
        pltpu.VMEM((CPW // 2, K), jnp.int32),
        pltpu.VMEM((K, H), jnp.float32),
        pltpu.VMEM((K, H), jnp.float32),
        pltpu.VMEM_SHARED((N_PAD, H), jnp.float32),
        pltpu.SemaphoreType.DMA,
        pltpu.SemaphoreType.DMA,
    ],
)
def _agg128(table_hbm, src_hbm, dst_hbm, zeros_hbm, out_hbm,
            src_v, dst_v, rows_a, rows_b, acc_sh, gsem_a, gsem_b):
  core = lax.axis_index("c")
  sub = lax.axis_index("s")
  w = core * NS + sub
  hc = CPW // 2
  pltpu.sync_copy(zeros_hbm, acc_sh.at[pl.ds(sub * RPS, RPS)])
  plsc.subcore_barrier()

  for half in range(2):
    pltpu.sync_copy(src_hbm.at[w, pl.ds(half * hc, hc)], src_v)
    pltpu.sync_copy(dst_hbm.at[w, pl.ds(half * hc, hc)], dst_v)

    @pl.loop(0, hc, step=2)
    def _(j):
      pltpu.async_copy(table_hbm.at[src_v.at[j]], rows_a, gsem_a)
      pltpu.async_copy(table_hbm.at[src_v.at[j + 1]], rows_b, gsem_b)
      pltpu.make_async_copy(table_hbm.at[pl.ds(0, K)], rows_a, gsem_a).wait()
      pltpu.sync_copy(rows_a, acc_sh.at[dst_v.at[j]], add=True)
      pltpu.make_async_copy(table_hbm.at[pl.ds(0, K)], rows_b, gsem_b).wait()
      pltpu.sync_copy(rows_b, acc_sh.at[dst_v.at[j + 1]], add=True)

  plsc.subcore_barrier()
  pltpu.sync_copy(acc_sh.at[pl.ds(sub * RPS, RPS)],
                  out_hbm.at[core, pl.ds(sub * RPS, RPS)])


@functools.partial(
    pl.kernel,
    out_type=jax.ShapeDtypeStruct((NW, NR, 128), jnp.float32),
    mesh=_mesh,
    scratch_types=[
        pltpu.VMEM((CPW, K), jnp.int32),
        pltpu.VMEM((NR, 128), jnp.float32),
    ],
    compiler_params=_sc_params,
)
def _hist(dst_hbm, zeros_hbm, out_hbm, dst_v, acc_v):
  core = lax.axis_index("c")
  sub = lax.axis_index("s")
  w = core * NS + sub
  pltpu.sync_copy(zeros_hbm, acc_v)
  pltpu.sync_copy(dst_hbm.at[w], dst_v)
  ones = jnp.full((16,), 1.0, jnp.float32)

  @pl.loop(0, CPW)
  def _(j):
    @pl.loop(0, K, step=16)
    def _(t):
      d = dst_v[j, pl.ds(t, 16)]
      plsc.addupdate_scatter(
          acc_v, [jax.lax.shift_right_logical(d, 7), d & 127], ones)

  pltpu.sync_copy(acc_v, out_hbm.at[w])


@functools.partial(
    pl.kernel,
    out_type=jax.ShapeDtypeStruct((NW, NR, 128), jnp.float32),
    mesh=_mesh,
    scratch_types=[
        pltpu.VMEM((CPW, K), jnp.int32),
        pltpu.VMEM((CPW, K), jnp.int32),
        pltpu.VMEM((NR, 128), jnp.float32),
        pltpu.VMEM((NR, 128), jnp.float32),
    ],
    compiler_params=_sc_params,
)
def _sagg(z_hbm, src_hbm, dst_hbm, zeros_hbm, out_hbm,
          src_v, dst_v, z_v, acc_v):
  core = lax.axis_index("c")
  sub = lax.axis_index("s")
  w = core * NS + sub
  pltpu.sync_copy(zeros_hbm, acc_v)
  pltpu.sync_copy(z_hbm, z_v)
  pltpu.sync_copy(src_hbm.at[w], src_v)
  pltpu.sync_copy(dst_hbm.at[w], dst_v)

  @pl.loop(0, CPW)
  def _(j):
    @pl.loop(0, K, step=16)
    def _(t):
      s = src_v[j, pl.ds(t, 16)]
      d = dst_v[j, pl.ds(t, 16)]
      vals = plsc.load_gather(
          z_v, [jax.lax.shift_right_logical(s, 7), s & 127])
      plsc.addupdate_scatter(
          acc_v, [jax.lax.shift_right_logical(d, 7), d & 127], vals)

  pltpu.sync_copy(acc_v, out_hbm.at[w])


RB = 2000


def _mm1_body(x_ref, w_ref, o_ref):
  x = x_ref[...]
  w = w_ref[...]
  xh = x.astype(jnp.bfloat16).astype(jnp.float32)
  xl = x - xh
  wh = w.astype(jnp.bfloat16).astype(jnp.float32)
  wl = w - wh
  dims = (((1,), (1,)), ((), ()))
  dot = functools.partial(
      lax.dot_general, dimension_numbers=dims,
      preferred_element_type=jnp.float32)
  o_ref[...] = dot(xh, wh) + dot(xh, wl) + dot(xl, wh)


_mm1 = pl.pallas_call(
    _mm1_body,
    grid=(N // RB,),
    in_specs=[pl.BlockSpec((RB, D), lambda i: (i, 0)),
              pl.BlockSpec((H, D), lambda i: (0, 0))],
    out_specs=pl.BlockSpec((RB, H), lambda i: (i, 0)),
    out_shape=jax.ShapeDtypeStruct((N, H), jnp.float32),
)


def _dinv_body(degp_ref, o_ref):
  deg = jnp.full((NR, 128), 1.0, jnp.float32)
  for i in range(NW):
    deg = deg + degp_ref[i]
  o_ref[...] = lax.rsqrt(deg)


_dinv = pl.pallas_call(
    _dinv_body,
    grid=(1,),
    in_specs=[pl.BlockSpec((NW, NR, 128), lambda i: (0, 0, 0))],
    out_specs=pl.BlockSpec((NR, 128), lambda i: (0, 0)),
    out_shape=jax.ShapeDtypeStruct((NR, 128), jnp.float32),
)


def _scale1_body(xw_ref, dinv_ref, o_y):
  o_y[...] = xw_ref[...] * dinv_ref[...]


_scale1 = pl.pallas_call(
    _scale1_body,
    grid=(N // RB,),
    in_specs=[pl.BlockSpec((RB, H), lambda i: (i, 0)),
              pl.BlockSpec((RB, 1), lambda i: (i, 0))],
    out_specs=pl.BlockSpec((RB, H), lambda i: (i, 0)),
    out_shape=jax.ShapeDtypeStruct((N, H), jnp.float32),
)


def _layer2_body(accp_ref, y_ref, dinv_ref, b1_ref, w2_ref, o_z):
  s = accp_ref[0] + accp_ref[1] + y_ref[...]
  h = jnp.maximum(s * dinv_ref[...] + b1_ref[...][None, :], 0.0)
  o_z[...] = jnp.sum(h * w2_ref[...], axis=1, keepdims=True) * dinv_ref[...]


_layer2 = pl.pallas_call(
    _layer2_body,
    grid=(N // RB,),
    in_specs=[pl.BlockSpec((2, RB, H), lambda i: (0, i, 0)),
              pl.BlockSpec((RB, H), lambda i: (i, 0)),
              pl.BlockSpec((RB, 1), lambda i: (i, 0)),
              pl.BlockSpec((H,), lambda i: (0,)),
              pl.BlockSpec((1, H), lambda i: (0, 0))],
    out_specs=pl.BlockSpec((RB, 1), lambda i: (i, 0)),
    out_shape=jax.ShapeDtypeStruct((N, 1), jnp.float32),
)


def _final_body(acc2p_ref, z_ref, dinv_ref, b2_ref, o_ref):
  s = z_ref[...]
  for i in range(NW):
    s = s + acc2p_ref[i]
  o_ref[...] = dinv_ref[...] * s + b2_ref[...]


_final = pl.pallas_call(
    _final_body,
    grid=(1,),
    in_specs=[pl.BlockSpec((NW, NR, 128), lambda i: (0, 0, 0)),
              pl.BlockSpec((NR, 128), lambda i: (0, 0)),
              pl.BlockSpec((NR, 128), lambda i: (0, 0)),
              pl.BlockSpec((1,), lambda i: (0,))],
    out_specs=pl.BlockSpec((NR, 128), lambda i: (0, 0)),
    out_shape=jax.ShapeDtypeStruct((NR, 128), jnp.float32),
)


def kernel(x, edge_index, W1, b1, W2, b2):
  src = edge_index[0]
  dst = edge_index[1]
  pad_idx = N + jnp.arange(E_PAD - E, dtype=jnp.int32) % (N_PAD - N)
  pad_src = pad_idx
  pad_dst = pad_idx
  src3 = jnp.concatenate([src, pad_src]).reshape(NW, CPW, K)
  dst3 = jnp.concatenate([dst, pad_dst]).reshape(NW, CPW, K)

  zeros_nr = jnp.zeros((NR, 128), jnp.float32)
  zeros128 = jnp.zeros((RPS, H), jnp.float32)

  degp = _hist(dst3, zeros_nr)
  xw = _mm1(x, W1)
  dinv80 = _dinv(degp)
  dinv = dinv80.reshape(N_PAD, 1)[:N]
  y = _scale1(xw, dinv)
  y_pad = jnp.concatenate(
      [y, jnp.zeros((N_PAD - N, H), jnp.float32)], axis=0)
  accp = _agg128(y_pad, src3, dst3, zeros128)
  z = _layer2(accp[:, :N, :], y, dinv, b1, W2)
  z80 = jnp.concatenate(
      [z[:, 0], jnp.zeros((N_PAD - N,), jnp.float32)]).reshape(NR, 128)
  acc2p = _sagg(z80, src3, dst3, zeros_nr)
  out80 = _final(acc2p, z80, dinv80, b2)
  return out80.reshape(N_PAD)[:N]


# speedup vs baseline: 1.3133x; 1.1952x over previous
import dataclasses
import functools

import jax
import jax.numpy as jnp
from jax import lax
from jax.experimental import pallas as pl
from jax.experimental.pallas import tpu as pltpu
from jax.experimental.pallas import tpu_sc as plsc

N = 10000
D = 128
H = 128
E = 320000

NC = 2
NS = 16
NW = NC * NS
K = 128
CPW = 80
G = 8
NG = CPW // G
E_PAD = NW * CPW * K
N_PAD = 10240
NR = N_PAD // 128
RPS = N_PAD // NS

_mesh = plsc.VectorSubcoreMesh(core_axis_name="c", subcore_axis_name="s")

_sc_params = pltpu.CompilerParams()
if "needs_layout_passes" in pltpu.CompilerParams.__dataclass_fields__:
  _sc_params = dataclasses.replace(_sc_params, needs_layout_passes=False)


@functools.partial(
    pl.kernel,
    out_type=jax.ShapeDtypeStruct((NC, N_PAD, H), jnp.float32),
    mesh=_mesh,
    scratch_types=[
        pltpu.VMEM((CPW // 2, K), jnp.int32),
        pltpu.VMEM((CPW // 2, K), jnp.int32),
        pltpu.VMEM((K, H), jnp.float32),
        pltpu.VMEM((K, H), jnp.float32),
        pltpu.VMEM_SHARED((N_PAD, H), jnp.float32),
        pltpu.SemaphoreType.DMA,
        pltpu.SemaphoreType.DMA,
    ],
)
def _agg128(table_hbm, src_hbm, dst_hbm, zeros_hbm, out_hbm,
            src_v, dst_v, rows_a, rows_b, acc_sh, gsem_a, gsem_b):
  core = lax.axis_index("c")
  sub = lax.axis_index("s")
  w = core * NS + sub
  hc = CPW // 2
  pltpu.sync_copy(zeros_hbm, acc_sh.at[pl.ds(sub * RPS, RPS)])
  plsc.subcore_barrier()

  for half in range(2):
    pltpu.sync_copy(src_hbm.at[w, pl.ds(half * hc, hc)], src_v)
    pltpu.sync_copy(dst_hbm.at[w, pl.ds(half * hc, hc)], dst_v)
    pltpu.async_copy(table_hbm.at[src_v.at[0]], rows_a, gsem_a)
    pltpu.async_copy(table_hbm.at[src_v.at[1]], rows_b, gsem_b)

    @pl.loop(0, hc - 2, step=2)
    def _(j):
      pltpu.make_async_copy(table_hbm.at[pl.ds(0, K)], rows_a, gsem_a).wait()
      pltpu.sync_copy(rows_a, acc_sh.at[dst_v.at[j]], add=True)
      pltpu.async_copy(table_hbm.at[src_v.at[j + 2]], rows_a, gsem_a)
      pltpu.make_async_copy(table_hbm.at[pl.ds(0, K)], rows_b, gsem_b).wait()
      pltpu.sync_copy(rows_b, acc_sh.at[dst_v.at[j + 1]], add=True)
      pltpu.async_copy(table_hbm.at[src_v.at[j + 3]], rows_b, gsem_b)

    pltpu.make_async_copy(table_hbm.at[pl.ds(0, K)], rows_a, gsem_a).wait()
    pltpu.sync_copy(rows_a, acc_sh.at[dst_v.at[hc - 2]], add=True)
    pltpu.make_async_copy(table_hbm.at[pl.ds(0, K)], rows_b, gsem_b).wait()
    pltpu.sync_copy(rows_b, acc_sh.at[dst_v.at[hc - 1]], add=True)

  plsc.subcore_barrier()
  pltpu.sync_copy(acc_sh.at[pl.ds(sub * RPS, RPS)],
                  out_hbm.at[core, pl.ds(sub * RPS, RPS)])


@functools.partial(
    pl.kernel,
    out_type=jax.ShapeDtypeStruct((NW, NR, 128), jnp.float32),
    mesh=_mesh,
    scratch_types=[
        pltpu.VMEM((CPW, K), jnp.int32),
        pltpu.VMEM((NR, 128), jnp.float32),
    ],
    compiler_params=_sc_params,
)
def _hist(dst_hbm, zeros_hbm, out_hbm, dst_v, acc_v):
  core = lax.axis_index("c")
  sub = lax.axis_index("s")
  w = core * NS + sub
  pltpu.sync_copy(zeros_hbm, acc_v)
  pltpu.sync_copy(dst_hbm.at[w], dst_v)
  ones = jnp.full((16,), 1.0, jnp.float32)

  @pl.loop(0, CPW)
  def _(j):
    @pl.loop(0, K, step=16)
    def _(t):
      d = dst_v[j, pl.ds(t, 16)]
      plsc.addupdate_scatter(
          acc_v, [jax.lax.shift_right_logical(d, 7), d & 127], ones)

  pltpu.sync_copy(acc_v, out_hbm.at[w])


@functools.partial(
    pl.kernel,
    out_type=jax.ShapeDtypeStruct((NW, NR, 128), jnp.float32),
    mesh=_mesh,
    scratch_types=[
        pltpu.VMEM((CPW, K), jnp.int32),
        pltpu.VMEM((CPW, K), jnp.int32),
        pltpu.VMEM((NR, 128), jnp.float32),
        pltpu.VMEM((NR, 128), jnp.float32),
    ],
    compiler_params=_sc_params,
)
def _sagg(z_hbm, src_hbm, dst_hbm, zeros_hbm, out_hbm,
          src_v, dst_v, z_v, acc_v):
  core = lax.axis_index("c")
  sub = lax.axis_index("s")
  w = core * NS + sub
  pltpu.sync_copy(zeros_hbm, acc_v)
  pltpu.sync_copy(z_hbm, z_v)
  pltpu.sync_copy(src_hbm.at[w], src_v)
  pltpu.sync_copy(dst_hbm.at[w], dst_v)

  @pl.loop(0, CPW)
  def _(j):
    @pl.loop(0, K, step=16)
    def _(t):
      s = src_v[j, pl.ds(t, 16)]
      d = dst_v[j, pl.ds(t, 16)]
      vals = plsc.load_gather(
          z_v, [jax.lax.shift_right_logical(s, 7), s & 127])
      plsc.addupdate_scatter(
          acc_v, [jax.lax.shift_right_logical(d, 7), d & 127], vals)

  pltpu.sync_copy(acc_v, out_hbm.at[w])


RB = 2000


def _mm1_body(x_ref, w_ref, o_ref):
  x = x_ref[...]
  w = w_ref[...]
  xh = x.astype(jnp.bfloat16).astype(jnp.float32)
  xl = x - xh
  wh = w.astype(jnp.bfloat16).astype(jnp.float32)
  wl = w - wh
  dims = (((1,), (1,)), ((), ()))
  dot = functools.partial(
      lax.dot_general, dimension_numbers=dims,
      preferred_element_type=jnp.float32)
  o_ref[...] = dot(xh, wh) + dot(xh, wl) + dot(xl, wh)


_mm1 = pl.pallas_call(
    _mm1_body,
    grid=(N // RB,),
    in_specs=[pl.BlockSpec((RB, D), lambda i: (i, 0)),
              pl.BlockSpec((H, D), lambda i: (0, 0))],
    out_specs=pl.BlockSpec((RB, H), lambda i: (i, 0)),
    out_shape=jax.ShapeDtypeStruct((N, H), jnp.float32),
)


def _dinv_body(degp_ref, o_ref):
  deg = jnp.full((NR, 128), 1.0, jnp.float32)
  for i in range(NW):
    deg = deg + degp_ref[i]
  o_ref[...] = lax.rsqrt(deg)


_dinv = pl.pallas_call(
    _dinv_body,
    grid=(1,),
    in_specs=[pl.BlockSpec((NW, NR, 128), lambda i: (0, 0, 0))],
    out_specs=pl.BlockSpec((NR, 128), lambda i: (0, 0)),
    out_shape=jax.ShapeDtypeStruct((NR, 128), jnp.float32),
)


def _scale1_body(xw_ref, dinv_ref, o_y):
  o_y[...] = xw_ref[...] * dinv_ref[...]


_scale1 = pl.pallas_call(
    _scale1_body,
    grid=(N // RB,),
    in_specs=[pl.BlockSpec((RB, H), lambda i: (i, 0)),
              pl.BlockSpec((RB, 1), lambda i: (i, 0))],
    out_specs=pl.BlockSpec((RB, H), lambda i: (i, 0)),
    out_shape=jax.ShapeDtypeStruct((N, H), jnp.float32),
)


def _layer2_body(accp_ref, y_ref, dinv_ref, b1_ref, w2_ref, o_z):
  s = accp_ref[0] + accp_ref[1] + y_ref[...]
  h = jnp.maximum(s * dinv_ref[...] + b1_ref[...][None, :], 0.0)
  o_z[...] = jnp.sum(h * w2_ref[...], axis=1, keepdims=True) * dinv_ref[...]


_layer2 = pl.pallas_call(
    _layer2_body,
    grid=(N // RB,),
    in_specs=[pl.BlockSpec((2, RB, H), lambda i: (0, i, 0)),
              pl.BlockSpec((RB, H), lambda i: (i, 0)),
              pl.BlockSpec((RB, 1), lambda i: (i, 0)),
              pl.BlockSpec((H,), lambda i: (0,)),
              pl.BlockSpec((1, H), lambda i: (0, 0))],
    out_specs=pl.BlockSpec((RB, 1), lambda i: (i, 0)),
    out_shape=jax.ShapeDtypeStruct((N, 1), jnp.float32),
)


def _final_body(acc2p_ref, z_ref, dinv_ref, b2_ref, o_ref):
  s = z_ref[...]
  for i in range(NW):
    s = s + acc2p_ref[i]
  o_ref[...] = dinv_ref[...] * s + b2_ref[...]


_final = pl.pallas_call(
    _final_body,
    grid=(1,),
    in_specs=[pl.BlockSpec((NW, NR, 128), lambda i: (0, 0, 0)),
              pl.BlockSpec((NR, 128), lambda i: (0, 0)),
              pl.BlockSpec((NR, 128), lambda i: (0, 0)),
              pl.BlockSpec((1,), lambda i: (0,))],
    out_specs=pl.BlockSpec((NR, 128), lambda i: (0, 0)),
    out_shape=jax.ShapeDtypeStruct((NR, 128), jnp.float32),
)


def kernel(x, edge_index, W1, b1, W2, b2):
  src = edge_index[0]
  dst = edge_index[1]
  pad_idx = N + jnp.arange(E_PAD - E, dtype=jnp.int32) % (N_PAD - N)
  pad_src = pad_idx
  pad_dst = pad_idx
  src3 = jnp.concatenate([src, pad_src]).reshape(NW, CPW, K)
  dst3 = jnp.concatenate([dst, pad_dst]).reshape(NW, CPW, K)

  zeros_nr = jnp.zeros((NR, 128), jnp.float32)
  zeros128 = jnp.zeros((RPS, H), jnp.float32)

  degp = _hist(dst3, zeros_nr)
  xw = _mm1(x, W1)
  dinv80 = _dinv(degp)
  dinv = dinv80.reshape(N_PAD, 1)[:N]
  y = _scale1(xw, dinv)
  y_pad = jnp.concatenate(
      [y, jnp.zeros((N_PAD - N, H), jnp.float32)], axis=0)
  accp = _agg128(y_pad, src3, dst3, zeros128)
  z = _layer2(accp[:, :N, :], y, dinv, b1, W2)
  z80 = jnp.concatenate(
      [z[:, 0], jnp.zeros((N_PAD - N,), jnp.float32)]).reshape(NR, 128)
  acc2p = _sagg(z80, src3, dst3, zeros_nr)
  out80 = _final(acc2p, z80, dinv80, b2)
  return out80.reshape(N_PAD)[:N]
